# baseline (speedup 1.0000x reference)
import functools

import jax
import jax.numpy as jnp
from jax import lax
from jax.experimental import pallas as pl
from jax.experimental.pallas import tpu as pltpu
from jax.experimental.pallas import tpu_sc as plsc

SEQ = 200
DIM = 64
BATCH = 4096
NC, NS, LANES = 2, 16, 16
NW = NC * NS
ROWS = BATCH * SEQ
RPW = ROWS // NW
CHUNK = 128
NCHUNK = RPW // CHUNK
NBUF = 4


def _body(table_hbm, idx_hbm, pos2_hbm, out_hbm, idx_v, pos2_v, rows_v,
          gsems, ssems):
    wid = lax.axis_index("s") * NC + lax.axis_index("c")
    base = wid * RPW

    pltpu.sync_copy(idx_hbm.at[wid], idx_v)
    pltpu.sync_copy(pos2_hbm, pos2_v)

    def issue_gather(jj, b):
        pltpu.async_copy(table_hbm.at[idx_v.at[jj]], rows_v.at[b],
                         gsems.at[b])

    def wait_gather(jj, b):
        pltpu.make_async_copy(table_hbm.at[idx_v.at[jj]], rows_v.at[b],
                              gsems.at[b]).wait()

    def issue_store(jj, b):
        pltpu.async_copy(rows_v.at[b],
                         out_hbm.at[pl.ds(base + jj * CHUNK, CHUNK)],
                         ssems.at[b])

    def wait_store(jj, b):
        pltpu.make_async_copy(rows_v.at[b],
                              out_hbm.at[pl.ds(base + jj * CHUNK, CHUNK)],
                              ssems.at[b]).wait()

    for jj in range(NBUF - 1):
        issue_gather(jj, jj)

    @pl.loop(0, NCHUNK, step=NBUF)
    def _outer(j):
        for b in range(NBUF):
            jj = j + b
            wait_gather(jj, b)
            start = lax.rem(jj * CHUNK, SEQ)

            @pl.loop(0, CHUNK)
            def _add(r):
                pr = start + r
                for c in range(DIM // LANES):
                    sl = pl.ds(c * LANES, LANES)
                    plsc.addupdate(rows_v.at[b, r, sl], pos2_v[pr, sl])

            issue_store(jj, b)

            nxt = jj + NBUF - 1
            bn = nxt % NBUF

            @pl.when(nxt < NCHUNK)
            def _prefetch():
                @pl.when(jj >= 1)
                def _drain():
                    wait_store(jj - 1, bn)

                issue_gather(nxt, bn)

    for jj in range(NCHUNK - NBUF, NCHUNK):
        wait_store(jj, jj % NBUF)


@jax.jit
def _pos_embed(idx3, term_table, pos2):
    mesh = plsc.VectorSubcoreMesh(core_axis_name="c", subcore_axis_name="s")
    run = pl.kernel(
        _body,
        out_type=jax.ShapeDtypeStruct((ROWS, DIM), jnp.float32),
        mesh=mesh,
        scratch_types=[
            pltpu.VMEM((NCHUNK, CHUNK), jnp.int32),
            pltpu.VMEM((2 * SEQ, DIM), jnp.float32),
            pltpu.VMEM((NBUF, CHUNK, DIM), jnp.float32),
            pltpu.SemaphoreType.DMA((NBUF,)),
            pltpu.SemaphoreType.DMA((NBUF,)),
        ],
        compiler_params=pltpu.CompilerParams(use_tc_tiling_on_sc=False),
    )
    return run(term_table, idx3, pos2)


def kernel(inputs, term_table, pos_table):
    idx3 = inputs.astype(jnp.int32).reshape(NW, NCHUNK, CHUNK)
    pos2 = jnp.concatenate([pos_table, pos_table], axis=0)
    out = _pos_embed(idx3, term_table, pos2)
    return out.reshape(BATCH, SEQ, DIM)


# speedup vs baseline: 1.2378x; 1.2378x over previous
import jax
import jax.numpy as jnp
from jax import lax
from jax.experimental import pallas as pl
from jax.experimental.pallas import tpu as pltpu
from jax.experimental.pallas import tpu_sc as plsc

SEQ = 200
DIM = 64
BATCH = 4096
NC, NS, LANES = 2, 16, 16
NW = NC * NS
BPW = BATCH // NW
GA = 128
GB = SEQ - GA
NBUF = 4


def _body(table_hbm, idx_hbm, pos_hbm, out_hbm, idx_v, pos_v, rows_v,
          gsems, ssems):
    wid = lax.axis_index("s") * NC + lax.axis_index("c")
    bbase = wid * BPW

    pltpu.sync_copy(idx_hbm.at[wid], idx_v)
    pltpu.sync_copy(pos_hbm, pos_v)

    def gathers(s, b):
        yield (table_hbm.at[idx_v.at[s, 0]],
               rows_v.at[b, pl.ds(0, GA)], gsems.at[b])
        yield (table_hbm.at[idx_v.at[s, 1, pl.ds(0, GB)]],
               rows_v.at[b, pl.ds(GA, GB)], gsems.at[b])

    def issue_gather(s, b):
        for src, dst, sem in gathers(s, b):
            pltpu.async_copy(src, dst, sem)

    def wait_gather(s, b):
        for src, dst, sem in gathers(s, b):
            pltpu.make_async_copy(src, dst, sem).wait()

    def issue_store(s, b):
        pltpu.async_copy(rows_v.at[b], out_hbm.at[bbase + s], ssems.at[b])

    def wait_store(s, b):
        pltpu.make_async_copy(rows_v.at[b], out_hbm.at[bbase + s],
                              ssems.at[b]).wait()

    for s in range(NBUF - 1):
        issue_gather(s, s)

    @pl.loop(0, BPW, step=NBUF)
    def _outer(j):
        for b in range(NBUF):
            s = j + b
            wait_gather(s, b)

            @pl.loop(0, SEQ)
            def _add(r):
                for c in range(DIM // LANES):
                    sl = pl.ds(c * LANES, LANES)
                    plsc.addupdate(rows_v.at[b, r, sl], pos_v[r, sl])

            issue_store(s, b)

            nxt = s + NBUF - 1
            bn = nxt % NBUF

            @pl.when(nxt < BPW)
            def _prefetch():
                @pl.when(s >= 1)
                def _drain():
                    wait_store(s - 1, bn)

                issue_gather(nxt, bn)

    for s in range(BPW - NBUF, BPW):
        wait_store(s, s % NBUF)


@jax.jit
def _pos_embed(idx4, term_table, pos_table):
    mesh = plsc.VectorSubcoreMesh(core_axis_name="c", subcore_axis_name="s")
    run = pl.kernel(
        _body,
        out_type=jax.ShapeDtypeStruct((BATCH, SEQ, DIM), jnp.float32),
        mesh=mesh,
        scratch_types=[
            pltpu.VMEM((BPW, 2, GA), jnp.int32),
            pltpu.VMEM((SEQ, DIM), jnp.float32),
            pltpu.VMEM((NBUF, SEQ, DIM), jnp.float32),
            pltpu.SemaphoreType.DMA((NBUF,)),
            pltpu.SemaphoreType.DMA((NBUF,)),
        ],
        compiler_params=pltpu.CompilerParams(use_tc_tiling_on_sc=False),
    )
    return run(term_table, idx4, pos_table)


def kernel(inputs, term_table, pos_table):
    idx = inputs.astype(jnp.int32).reshape(NW, BPW, SEQ)
    idx4 = jnp.stack(
        [idx[:, :, :GA], jnp.pad(idx[:, :, GA:], ((0, 0), (0, 0), (0, GA - GB)))],
        axis=2)
    return _pos_embed(idx4, term_table, pos_table)
